# baseline (speedup 1.0000x reference)
import functools
import jax
import jax.numpy as jnp
from jax import lax
from jax.experimental import pallas as pl
from jax.experimental.pallas import tpu as pltpu
from jax.experimental.pallas import tpu_sc as plsc

BATCH = 1024
K = 100000
LANES = 16
NWORKERS = 32
ROWS_PER_W = BATCH // NWORKERS
RG = 8
NGROUPS = ROWS_PER_W // RG
CH = 2000
NCHUNK = K // CH


def _sc_body(state_hbm, values_hbm, out_hbm, vals_v, buf_v, wide_v, out_v):
    wid = lax.axis_index("s") * 2 + lax.axis_index("c")
    rbase = wid * ROWS_PER_W

    pltpu.sync_copy(values_hbm, vals_v)

    def group_body(g, carry):
        row0 = rbase + g * RG

        def chunk_body(c, accs):
            k0 = c * CH
            pltpu.sync_copy(state_hbm.at[pl.ds(row0, RG), pl.ds(k0, CH)],
                            buf_v)

            def inner(j, accs):
                col = j * LANES
                v = vals_v[pl.ds(k0 + col, LANES)]
                return tuple(
                    accs[rr] + buf_v[rr, pl.ds(col, LANES)] * v
                    for rr in range(RG)
                )

            return lax.fori_loop(0, CH // LANES, inner, accs)

        zero = jnp.zeros((LANES,), jnp.float32)
        accs = lax.fori_loop(0, NCHUNK, chunk_body, (zero,) * RG)

        for rr in range(RG):
            wide_v[pl.ds((g * RG + rr) * LANES, LANES)] = accs[rr]
        return carry

    lax.fori_loop(0, NGROUPS, group_body, 0)

    lane_ids = lax.iota(jnp.int32, LANES)
    for half in range(2):
        tot = jnp.zeros((LANES,), jnp.float32)
        for rr in range(LANES):
            r = half * LANES + rr
            v = wide_v[pl.ds(r * LANES, LANES)]
            s = v[0]
            for p in range(1, LANES):
                s = s + v[p]
            tot = jnp.where(lane_ids == rr, s, tot)
        out_v[pl.ds(half * LANES, LANES)] = tot
    pltpu.sync_copy(out_v, out_hbm.at[pl.ds(rbase, ROWS_PER_W)])


@jax.jit
def _matvec(state, values_flat):
    mesh = plsc.VectorSubcoreMesh(
        core_axis_name="c", subcore_axis_name="s",
        num_cores=2, num_subcores=16,
    )
    f = pl.kernel(
        _sc_body,
        out_type=jax.ShapeDtypeStruct((BATCH,), jnp.float32),
        mesh=mesh,
        scratch_types=[
            pltpu.VMEM((K,), jnp.float32),
            pltpu.VMEM((RG, CH), jnp.float32),
            pltpu.VMEM((ROWS_PER_W * LANES,), jnp.float32),
            pltpu.VMEM((ROWS_PER_W,), jnp.float32),
        ],
        compiler_params=pltpu.CompilerParams(use_tc_tiling_on_sc=False),
    )
    return f(state, values_flat)


def kernel(state, values):
    out = _matvec(state, values.reshape(K))
    return out.reshape(BATCH, 1)


# speedup vs baseline: 1.9743x; 1.9743x over previous
import functools
import jax
import jax.numpy as jnp
from jax import lax
from jax.experimental import pallas as pl
from jax.experimental.pallas import tpu as pltpu
from jax.experimental.pallas import tpu_sc as plsc

BATCH = 1024
K = 100000
LANES = 16
NWORKERS = 32
ROWS_PER_W = BATCH // NWORKERS
RG = 8
NGROUPS = ROWS_PER_W // RG
CH = 1024
NCHUNK = K // CH
KSC = NCHUNK * CH
KTAIL = K - KSC


def _sc_body(state_hbm, values_hbm, out_hbm, vals_v, buf0_v, buf1_v, wide_v,
             out_v, sem0, sem1):
    wid = lax.axis_index("s") * 2 + lax.axis_index("c")
    rbase = wid * ROWS_PER_W

    pltpu.sync_copy(values_hbm, vals_v)

    def fma_chunk(buf, c, accs):
        k0 = c * CH

        def inner(j, accs):
            col = j * LANES
            v = vals_v[pl.ds(k0 + col, LANES)]
            return tuple(
                accs[rr] + buf[rr, pl.ds(col, LANES)] * v
                for rr in range(RG)
            )

        return lax.fori_loop(0, CH // LANES, inner, accs, unroll=4)

    def start_dma(row0, c, buf, sem):
        k0 = pl.multiple_of(c * CH, 128)
        return pltpu.async_copy(
            state_hbm.at[pl.ds(row0, RG), pl.ds(k0, CH)], buf, sem)

    def group_body(g, carry):
        row0 = pl.multiple_of(rbase + g * RG, RG)

        start_dma(row0, 0, buf0_v, sem0)

        def pair_body(i, accs):
            c0 = i * 2
            start_dma(row0, c0 + 1, buf1_v, sem1)
            pltpu.make_async_copy(
                state_hbm.at[pl.ds(row0, RG), pl.ds(0, CH)], buf0_v,
                sem0).wait()
            accs = fma_chunk(buf0_v, c0, accs)
            start_dma(row0, c0 + 2, buf0_v, sem0)
            pltpu.make_async_copy(
                state_hbm.at[pl.ds(row0, RG), pl.ds(0, CH)], buf1_v,
                sem1).wait()
            return fma_chunk(buf1_v, c0 + 1, accs)

        zero = jnp.zeros((LANES,), jnp.float32)
        accs = lax.fori_loop(0, (NCHUNK - 1) // 2, pair_body, (zero,) * RG)

        pltpu.make_async_copy(
            state_hbm.at[pl.ds(row0, RG), pl.ds(0, CH)], buf0_v, sem0).wait()
        accs = fma_chunk(buf0_v, NCHUNK - 1, accs)

        for rr in range(RG):
            wide_v[pl.ds((g * RG + rr) * LANES, LANES)] = accs[rr]
        return carry

    lax.fori_loop(0, NGROUPS, group_body, 0)

    lane_ids = lax.iota(jnp.int32, LANES)
    for half in range(2):
        tot = jnp.zeros((LANES,), jnp.float32)
        for rr in range(LANES):
            r = half * LANES + rr
            v = wide_v[pl.ds(r * LANES, LANES)]
            s = v[0]
            for p in range(1, LANES):
                s = s + v[p]
            tot = jnp.where(lane_ids == rr, s, tot)
        out_v[0, pl.ds(half * LANES, LANES)] = tot
    pltpu.sync_copy(out_v, out_hbm.at[wid])


def _tc_tail_body(state_tail_ref, vals_tail_ref, part_ref, out_ref):
    out_ref[...] = part_ref[...] + jnp.dot(
        state_tail_ref[...], vals_tail_ref[...],
        preferred_element_type=jnp.float32)


@jax.jit
def _matvec(state, values):
    mesh = plsc.VectorSubcoreMesh(
        core_axis_name="c", subcore_axis_name="s",
        num_cores=2, num_subcores=16,
    )
    sc_fn = pl.kernel(
        _sc_body,
        out_type=jax.ShapeDtypeStruct((NWORKERS, 1, ROWS_PER_W), jnp.float32),
        mesh=mesh,
        scratch_types=[
            pltpu.VMEM((KSC,), jnp.float32),
            pltpu.VMEM((RG, CH), jnp.float32),
            pltpu.VMEM((RG, CH), jnp.float32),
            pltpu.VMEM((ROWS_PER_W * LANES,), jnp.float32),
            pltpu.VMEM((1, ROWS_PER_W), jnp.float32),
            pltpu.SemaphoreType.DMA,
            pltpu.SemaphoreType.DMA,
        ],
    )
    vals_flat = values.reshape(K)
    out_sc = sc_fn(state, vals_flat[:KSC])
    part = out_sc.reshape(BATCH, 1)

    state_tail = lax.slice(state, (0, KSC), (BATCH, K))
    vals_tail = lax.slice(values, (KSC, 0), (K, 1))
    out = pl.pallas_call(
        _tc_tail_body,
        out_shape=jax.ShapeDtypeStruct((BATCH, 1), jnp.float32),
    )(state_tail, vals_tail, part)
    return out


def kernel(state, values):
    return _matvec(state, values)


# speedup vs baseline: 2.2827x; 1.1562x over previous
import functools
import jax
import jax.numpy as jnp
from jax import lax
from jax.experimental import pallas as pl
from jax.experimental.pallas import tpu as pltpu
from jax.experimental.pallas import tpu_sc as plsc

BATCH = 1024
K = 100000
LANES = 16
NWORKERS = 32
ROWS_PER_W = BATCH // NWORKERS
PR = 16
NPASS = ROWS_PER_W // PR
CH = 3072
NCHUNK = 32
KSC = NCHUNK * CH
TCB = 512
TC_FIRST_BLK = KSC // TCB
TC_NBLK = (K - KSC + TCB - 1) // TCB


def _sc_body(state_hbm, values_hbm, out_hbm, buf0, buf1, vb0, vb1, wide_v,
             out_v, sem0, sem1, vsem0, vsem1):
    wid = lax.axis_index("s") * 2 + lax.axis_index("c")
    rbase = wid * ROWS_PER_W

    def start_state(row0, c, buf, sem):
        k0 = pl.multiple_of(c * CH, 128)
        pltpu.async_copy(
            state_hbm.at[pl.ds(row0, PR), pl.ds(k0, CH)], buf, sem)

    def start_vals(c, vb, vsem):
        k0 = pl.multiple_of(c * CH, 128)
        pltpu.async_copy(values_hbm.at[pl.ds(k0, CH)], vb, vsem)

    def wait_state(row0, buf, sem):
        pltpu.make_async_copy(
            state_hbm.at[pl.ds(row0, PR), pl.ds(0, CH)], buf, sem).wait()

    def wait_vals(vb, vsem):
        pltpu.make_async_copy(values_hbm.at[pl.ds(0, CH)], vb, vsem).wait()

    def fma_chunk(buf, vb, accs):
        def inner(j, accs):
            col = j * LANES
            v = vb[pl.ds(col, LANES)]
            return tuple(
                accs[rr] + buf[rr, pl.ds(col, LANES)] * v
                for rr in range(PR)
            )
        return lax.fori_loop(0, CH // LANES, inner, accs, unroll=2)

    for ps in range(NPASS):
        row0 = pl.multiple_of(rbase + ps * PR, 8)

        start_state(row0, 0, buf0, sem0)
        start_vals(0, vb0, vsem0)

        def pair_body(i, accs, row0=row0):
            c = i * 2
            start_state(row0, c + 1, buf1, sem1)
            start_vals(c + 1, vb1, vsem1)
            wait_state(row0, buf0, sem0)
            wait_vals(vb0, vsem0)
            accs = fma_chunk(buf0, vb0, accs)
            start_state(row0, c + 2, buf0, sem0)
            start_vals(c + 2, vb0, vsem0)
            wait_state(row0, buf1, sem1)
            wait_vals(vb1, vsem1)
            return fma_chunk(buf1, vb1, accs)

        zero = jnp.zeros((LANES,), jnp.float32)
        accs = lax.fori_loop(0, NCHUNK // 2 - 1, pair_body, (zero,) * PR)

        start_state(row0, NCHUNK - 1, buf1, sem1)
        start_vals(NCHUNK - 1, vb1, vsem1)
        wait_state(row0, buf0, sem0)
        wait_vals(vb0, vsem0)
        accs = fma_chunk(buf0, vb0, accs)
        wait_state(row0, buf1, sem1)
        wait_vals(vb1, vsem1)
        accs = fma_chunk(buf1, vb1, accs)

        for rr in range(PR):
            wide_v[pl.ds((ps * PR + rr) * LANES, LANES)] = accs[rr]

    lane_ids = lax.iota(jnp.int32, LANES)
    for half in range(2):
        tot = jnp.zeros((LANES,), jnp.float32)
        for rr in range(LANES):
            r = half * LANES + rr
            v = wide_v[pl.ds(r * LANES, LANES)]
            s = v[0]
            for p in range(1, LANES):
                s = s + v[p]
            tot = jnp.where(lane_ids == rr, s, tot)
        out_v[0, pl.ds(half * LANES, LANES)] = tot
    pltpu.sync_copy(out_v, out_hbm.at[wid])


def _tc_suffix_body(state_ref, vals_ref, out_ref):
    c = pl.program_id(0)
    col0 = (TC_FIRST_BLK + c) * TCB
    cols = col0 + lax.broadcasted_iota(jnp.int32, (1, TCB), 1)
    valid = cols < K
    vals = jnp.where(valid, vals_ref[...], 0.0)
    st = jnp.where(valid, state_ref[...], 0.0)
    part = jnp.sum(st * vals, axis=1, keepdims=True)

    @pl.when(c == 0)
    def _():
        out_ref[...] = part

    @pl.when(c != 0)
    def _():
        out_ref[...] += part


def _tc_add_body(a_ref, b_ref, out_ref):
    out_ref[...] = a_ref[...] + b_ref[...]


@jax.jit
def _matvec(state, values):
    mesh = plsc.VectorSubcoreMesh(
        core_axis_name="c", subcore_axis_name="s",
        num_cores=2, num_subcores=16,
    )
    sc_fn = pl.kernel(
        _sc_body,
        out_type=jax.ShapeDtypeStruct((NWORKERS, 1, ROWS_PER_W), jnp.float32),
        mesh=mesh,
        scratch_types=[
            pltpu.VMEM((PR, CH), jnp.float32),
            pltpu.VMEM((PR, CH), jnp.float32),
            pltpu.VMEM((CH,), jnp.float32),
            pltpu.VMEM((CH,), jnp.float32),
            pltpu.VMEM((ROWS_PER_W * LANES,), jnp.float32),
            pltpu.VMEM((1, ROWS_PER_W), jnp.float32),
            pltpu.SemaphoreType.DMA,
            pltpu.SemaphoreType.DMA,
            pltpu.SemaphoreType.DMA,
            pltpu.SemaphoreType.DMA,
        ],
    )
    vals_flat = values.reshape(K)
    out_sc = sc_fn(state, vals_flat[:KSC])
    part_sc = out_sc.reshape(BATCH, 1)

    part_tc = pl.pallas_call(
        _tc_suffix_body,
        grid=(TC_NBLK,),
        in_specs=[
            pl.BlockSpec((BATCH, TCB), lambda c: (0, TC_FIRST_BLK + c)),
            pl.BlockSpec((1, TCB), lambda c: (0, TC_FIRST_BLK + c)),
        ],
        out_specs=pl.BlockSpec((BATCH, 1), lambda c: (0, 0)),
        out_shape=jax.ShapeDtypeStruct((BATCH, 1), jnp.float32),
    )(state, values.reshape(1, K))

    out = pl.pallas_call(
        _tc_add_body,
        out_shape=jax.ShapeDtypeStruct((BATCH, 1), jnp.float32),
    )(part_sc, part_tc)
    return out


def kernel(state, values):
    return _matvec(state, values)


# speedup vs baseline: 2.2941x; 1.0050x over previous
import functools
import jax
import jax.numpy as jnp
from jax import lax
from jax.experimental import pallas as pl
from jax.experimental.pallas import tpu as pltpu
from jax.experimental.pallas import tpu_sc as plsc

BATCH = 1024
K = 100000
LANES = 16
NWORKERS = 32
ROWS_PER_W = BATCH // NWORKERS
PR = 16
NPASS = ROWS_PER_W // PR
CH = 3072
NCHUNK = 8
KSC = NCHUNK * CH
TCB = 512
TC_FIRST_BLK = KSC // TCB
TC_NBLK = (K - KSC + TCB - 1) // TCB


def _sc_body(state_hbm, values_hbm, out_hbm, buf0, buf1, vb0, vb1, wide_v,
             out_v, sem0, sem1, vsem0, vsem1):
    wid = lax.axis_index("s") * 2 + lax.axis_index("c")
    rbase = wid * ROWS_PER_W

    def start_state(row0, c, buf, sem):
        k0 = pl.multiple_of(c * CH, 128)
        pltpu.async_copy(
            state_hbm.at[pl.ds(row0, PR), pl.ds(k0, CH)], buf, sem)

    def start_vals(c, vb, vsem):
        k0 = pl.multiple_of(c * CH, 128)
        pltpu.async_copy(values_hbm.at[pl.ds(k0, CH)], vb, vsem)

    def wait_state(row0, buf, sem):
        pltpu.make_async_copy(
            state_hbm.at[pl.ds(row0, PR), pl.ds(0, CH)], buf, sem).wait()

    def wait_vals(vb, vsem):
        pltpu.make_async_copy(values_hbm.at[pl.ds(0, CH)], vb, vsem).wait()

    def fma_chunk(buf, vb, accs):
        def inner(j, accs):
            col = j * LANES
            v = vb[pl.ds(col, LANES)]
            return tuple(
                accs[rr] + buf[rr, pl.ds(col, LANES)] * v
                for rr in range(PR)
            )
        return lax.fori_loop(0, CH // LANES, inner, accs, unroll=2)

    for ps in range(NPASS):
        row0 = pl.multiple_of(rbase + ps * PR, 8)

        start_state(row0, 0, buf0, sem0)
        start_vals(0, vb0, vsem0)

        def pair_body(i, accs, row0=row0):
            c = i * 2
            start_state(row0, c + 1, buf1, sem1)
            start_vals(c + 1, vb1, vsem1)
            wait_state(row0, buf0, sem0)
            wait_vals(vb0, vsem0)
            accs = fma_chunk(buf0, vb0, accs)
            start_state(row0, c + 2, buf0, sem0)
            start_vals(c + 2, vb0, vsem0)
            wait_state(row0, buf1, sem1)
            wait_vals(vb1, vsem1)
            return fma_chunk(buf1, vb1, accs)

        zero = jnp.zeros((LANES,), jnp.float32)
        accs = lax.fori_loop(0, NCHUNK // 2 - 1, pair_body, (zero,) * PR)

        start_state(row0, NCHUNK - 1, buf1, sem1)
        start_vals(NCHUNK - 1, vb1, vsem1)
        wait_state(row0, buf0, sem0)
        wait_vals(vb0, vsem0)
        accs = fma_chunk(buf0, vb0, accs)
        wait_state(row0, buf1, sem1)
        wait_vals(vb1, vsem1)
        accs = fma_chunk(buf1, vb1, accs)

        for rr in range(PR):
            wide_v[pl.ds((ps * PR + rr) * LANES, LANES)] = accs[rr]

    lane_ids = lax.iota(jnp.int32, LANES)
    for half in range(2):
        tot = jnp.zeros((LANES,), jnp.float32)
        for rr in range(LANES):
            r = half * LANES + rr
            v = wide_v[pl.ds(r * LANES, LANES)]
            s = v[0]
            for p in range(1, LANES):
                s = s + v[p]
            tot = jnp.where(lane_ids == rr, s, tot)
        out_v[0, pl.ds(half * LANES, LANES)] = tot
    pltpu.sync_copy(out_v, out_hbm.at[wid])


def _tc_suffix_body(state_ref, vals_ref, out_ref):
    c = pl.program_id(0)
    col0 = (TC_FIRST_BLK + c) * TCB
    cols = col0 + lax.broadcasted_iota(jnp.int32, (1, TCB), 1)
    valid = cols < K
    vals = jnp.where(valid, vals_ref[...], 0.0)
    st = jnp.where(valid, state_ref[...], 0.0)
    part = jnp.sum(st * vals, axis=1, keepdims=True)

    @pl.when(c == 0)
    def _():
        out_ref[...] = part

    @pl.when(c != 0)
    def _():
        out_ref[...] += part


def _tc_add_body(a_ref, b_ref, out_ref):
    out_ref[...] = a_ref[...] + b_ref[...]


@jax.jit
def _matvec(state, values):
    mesh = plsc.VectorSubcoreMesh(
        core_axis_name="c", subcore_axis_name="s",
        num_cores=2, num_subcores=16,
    )
    sc_fn = pl.kernel(
        _sc_body,
        out_type=jax.ShapeDtypeStruct((NWORKERS, 1, ROWS_PER_W), jnp.float32),
        mesh=mesh,
        scratch_types=[
            pltpu.VMEM((PR, CH), jnp.float32),
            pltpu.VMEM((PR, CH), jnp.float32),
            pltpu.VMEM((CH,), jnp.float32),
            pltpu.VMEM((CH,), jnp.float32),
            pltpu.VMEM((ROWS_PER_W * LANES,), jnp.float32),
            pltpu.VMEM((1, ROWS_PER_W), jnp.float32),
            pltpu.SemaphoreType.DMA,
            pltpu.SemaphoreType.DMA,
            pltpu.SemaphoreType.DMA,
            pltpu.SemaphoreType.DMA,
        ],
    )
    vals_flat = values.reshape(K)
    out_sc = sc_fn(state, vals_flat[:KSC])
    part_sc = out_sc.reshape(BATCH, 1)

    part_tc = pl.pallas_call(
        _tc_suffix_body,
        grid=(TC_NBLK,),
        in_specs=[
            pl.BlockSpec((BATCH, TCB), lambda c: (0, TC_FIRST_BLK + c)),
            pl.BlockSpec((1, TCB), lambda c: (0, TC_FIRST_BLK + c)),
        ],
        out_specs=pl.BlockSpec((BATCH, 1), lambda c: (0, 0)),
        out_shape=jax.ShapeDtypeStruct((BATCH, 1), jnp.float32),
    )(state, values.reshape(1, K))

    out = pl.pallas_call(
        _tc_add_body,
        out_shape=jax.ShapeDtypeStruct((BATCH, 1), jnp.float32),
    )(part_sc, part_tc)
    return out


def kernel(state, values):
    return _matvec(state, values)


# speedup vs baseline: 2.3314x; 1.0162x over previous
import functools
import jax
import jax.numpy as jnp
from jax import lax
from jax.experimental import pallas as pl
from jax.experimental.pallas import tpu as pltpu
from jax.experimental.pallas import tpu_sc as plsc

BATCH = 1024
K = 100000
LANES = 16
NWORKERS = 32
ROWS_PER_W = BATCH // NWORKERS
PR = 8
NPASS = ROWS_PER_W // PR
CH = 3072
NCHUNK = 8
KSC = NCHUNK * CH
TCB = 512
TC_FIRST_BLK = KSC // TCB
TC_NBLK = (K - KSC + TCB - 1) // TCB


def _sc_body(state_hbm, values_hbm, out_hbm, buf0, buf1, vb0, vb1, wide_v,
             out_v, sem0, sem1, vsem0, vsem1):
    wid = lax.axis_index("s") * 2 + lax.axis_index("c")
    rbase = wid * ROWS_PER_W

    def start_state(row0, c, buf, sem):
        k0 = pl.multiple_of(c * CH, 128)
        pltpu.async_copy(
            state_hbm.at[pl.ds(row0, PR), pl.ds(k0, CH)], buf, sem)

    def start_vals(c, vb, vsem):
        k0 = pl.multiple_of(c * CH, 128)
        pltpu.async_copy(values_hbm.at[pl.ds(k0, CH)], vb, vsem)

    def wait_state(row0, buf, sem):
        pltpu.make_async_copy(
            state_hbm.at[pl.ds(row0, PR), pl.ds(0, CH)], buf, sem).wait()

    def wait_vals(vb, vsem):
        pltpu.make_async_copy(values_hbm.at[pl.ds(0, CH)], vb, vsem).wait()

    def fma_chunk(buf, vb, accs):
        def inner(j, accs):
            col = j * LANES
            v = vb[pl.ds(col, LANES)]
            return tuple(
                accs[rr] + buf[rr, pl.ds(col, LANES)] * v
                for rr in range(PR)
            )
        return lax.fori_loop(0, CH // LANES, inner, accs, unroll=2)

    for ps in range(NPASS):
        row0 = pl.multiple_of(rbase + ps * PR, 8)

        start_state(row0, 0, buf0, sem0)
        start_vals(0, vb0, vsem0)

        def pair_body(i, accs, row0=row0):
            c = i * 2
            start_state(row0, c + 1, buf1, sem1)
            start_vals(c + 1, vb1, vsem1)
            wait_state(row0, buf0, sem0)
            wait_vals(vb0, vsem0)
            accs = fma_chunk(buf0, vb0, accs)
            start_state(row0, c + 2, buf0, sem0)
            start_vals(c + 2, vb0, vsem0)
            wait_state(row0, buf1, sem1)
            wait_vals(vb1, vsem1)
            return fma_chunk(buf1, vb1, accs)

        zero = jnp.zeros((LANES,), jnp.float32)
        accs = lax.fori_loop(0, NCHUNK // 2 - 1, pair_body, (zero,) * PR)

        start_state(row0, NCHUNK - 1, buf1, sem1)
        start_vals(NCHUNK - 1, vb1, vsem1)
        wait_state(row0, buf0, sem0)
        wait_vals(vb0, vsem0)
        accs = fma_chunk(buf0, vb0, accs)
        wait_state(row0, buf1, sem1)
        wait_vals(vb1, vsem1)
        accs = fma_chunk(buf1, vb1, accs)

        for rr in range(PR):
            wide_v[pl.ds((ps * PR + rr) * LANES, LANES)] = accs[rr]

    lane_ids = lax.iota(jnp.int32, LANES)
    for half in range(2):
        tot = jnp.zeros((LANES,), jnp.float32)
        for rr in range(LANES):
            r = half * LANES + rr
            v = wide_v[pl.ds(r * LANES, LANES)]
            s = v[0]
            for p in range(1, LANES):
                s = s + v[p]
            tot = jnp.where(lane_ids == rr, s, tot)
        out_v[0, pl.ds(half * LANES, LANES)] = tot
    pltpu.sync_copy(out_v, out_hbm.at[wid])


def _tc_suffix_body(state_ref, vals_ref, out_ref):
    c = pl.program_id(0)

    @pl.when(c == 0)
    def _():
        out_ref[...] = jnp.zeros_like(out_ref)

    @pl.when(c < TC_NBLK - 1)
    def _():
        out_ref[...] += jnp.sum(state_ref[...] * vals_ref[...], axis=1,
                                keepdims=True)

    @pl.when(c == TC_NBLK - 1)
    def _():
        col0 = (TC_FIRST_BLK + c) * TCB
        cols = col0 + lax.broadcasted_iota(jnp.int32, (1, TCB), 1)
        valid = cols < K
        vals = jnp.where(valid, vals_ref[...], 0.0)
        st = jnp.where(valid, state_ref[...], 0.0)
        out_ref[...] += jnp.sum(st * vals, axis=1, keepdims=True)


def _tc_add_body(a_ref, b_ref, out_ref):
    out_ref[...] = a_ref[...] + b_ref[...]


@jax.jit
def _matvec(state, values):
    mesh = plsc.VectorSubcoreMesh(
        core_axis_name="c", subcore_axis_name="s",
        num_cores=2, num_subcores=16,
    )
    sc_fn = pl.kernel(
        _sc_body,
        out_type=jax.ShapeDtypeStruct((NWORKERS, 1, ROWS_PER_W), jnp.float32),
        mesh=mesh,
        scratch_types=[
            pltpu.VMEM((PR, CH), jnp.float32),
            pltpu.VMEM((PR, CH), jnp.float32),
            pltpu.VMEM((CH,), jnp.float32),
            pltpu.VMEM((CH,), jnp.float32),
            pltpu.VMEM((ROWS_PER_W * LANES,), jnp.float32),
            pltpu.VMEM((1, ROWS_PER_W), jnp.float32),
            pltpu.SemaphoreType.DMA,
            pltpu.SemaphoreType.DMA,
            pltpu.SemaphoreType.DMA,
            pltpu.SemaphoreType.DMA,
        ],
    )
    vals_flat = values.reshape(K)
    out_sc = sc_fn(state, vals_flat[:KSC])
    part_sc = out_sc.reshape(BATCH, 1)

    part_tc = pl.pallas_call(
        _tc_suffix_body,
        grid=(TC_NBLK,),
        in_specs=[
            pl.BlockSpec((BATCH, TCB), lambda c: (0, TC_FIRST_BLK + c)),
            pl.BlockSpec((1, TCB), lambda c: (0, TC_FIRST_BLK + c)),
        ],
        out_specs=pl.BlockSpec((BATCH, 1), lambda c: (0, 0)),
        out_shape=jax.ShapeDtypeStruct((BATCH, 1), jnp.float32),
    )(state, values.reshape(1, K))

    out = pl.pallas_call(
        _tc_add_body,
        out_shape=jax.ShapeDtypeStruct((BATCH, 1), jnp.float32),
    )(part_sc, part_tc)
    return out


def kernel(state, values):
    return _matvec(state, values)


# speedup vs baseline: 2.4201x; 1.0381x over previous
import functools
import jax
import jax.numpy as jnp
from jax import lax
from jax.experimental import pallas as pl
from jax.experimental.pallas import tpu as pltpu
from jax.experimental.pallas import tpu_sc as plsc

BATCH = 1024
K = 100000
LANES = 16
NWORKERS = 32
ROWS_PER_W = BATCH // NWORKERS
PR = 8
NPASS = ROWS_PER_W // PR
CH = 3072
NCHUNK = 8
KSC = NCHUNK * CH
TCB = 2048
TC_FIRST_BLK = KSC // TCB
TC_NBLK = (K - KSC + TCB - 1) // TCB


def _sc_body(state_hbm, values_hbm, out_hbm, buf0, buf1, vb0, vb1, wide_v,
             out_v, sem0, sem1, vsem0, vsem1):
    wid = lax.axis_index("s") * 2 + lax.axis_index("c")
    rbase = wid * ROWS_PER_W

    def start_state(row0, c, buf, sem):
        k0 = pl.multiple_of(c * CH, 128)
        pltpu.async_copy(
            state_hbm.at[pl.ds(row0, PR), pl.ds(k0, CH)], buf, sem)

    def start_vals(c, vb, vsem):
        k0 = pl.multiple_of(c * CH, 128)
        pltpu.async_copy(values_hbm.at[pl.ds(k0, CH)], vb, vsem)

    def wait_state(row0, buf, sem):
        pltpu.make_async_copy(
            state_hbm.at[pl.ds(row0, PR), pl.ds(0, CH)], buf, sem).wait()

    def wait_vals(vb, vsem):
        pltpu.make_async_copy(values_hbm.at[pl.ds(0, CH)], vb, vsem).wait()

    def fma_chunk(buf, vb, accs):
        def inner(j, accs):
            col = j * LANES
            v = vb[pl.ds(col, LANES)]
            return tuple(
                accs[rr] + buf[rr, pl.ds(col, LANES)] * v
                for rr in range(PR)
            )
        return lax.fori_loop(0, CH // LANES, inner, accs, unroll=2)

    for ps in range(NPASS):
        row0 = pl.multiple_of(rbase + ps * PR, 8)

        start_state(row0, 0, buf0, sem0)
        start_vals(0, vb0, vsem0)

        def pair_body(i, accs, row0=row0):
            c = i * 2
            start_state(row0, c + 1, buf1, sem1)
            start_vals(c + 1, vb1, vsem1)
            wait_state(row0, buf0, sem0)
            wait_vals(vb0, vsem0)
            accs = fma_chunk(buf0, vb0, accs)
            start_state(row0, c + 2, buf0, sem0)
            start_vals(c + 2, vb0, vsem0)
            wait_state(row0, buf1, sem1)
            wait_vals(vb1, vsem1)
            return fma_chunk(buf1, vb1, accs)

        zero = jnp.zeros((LANES,), jnp.float32)
        accs = lax.fori_loop(0, NCHUNK // 2 - 1, pair_body, (zero,) * PR)

        start_state(row0, NCHUNK - 1, buf1, sem1)
        start_vals(NCHUNK - 1, vb1, vsem1)
        wait_state(row0, buf0, sem0)
        wait_vals(vb0, vsem0)
        accs = fma_chunk(buf0, vb0, accs)
        wait_state(row0, buf1, sem1)
        wait_vals(vb1, vsem1)
        accs = fma_chunk(buf1, vb1, accs)

        for rr in range(PR):
            wide_v[pl.ds((ps * PR + rr) * LANES, LANES)] = accs[rr]

    lane_ids = lax.iota(jnp.int32, LANES)
    for half in range(2):
        tot = jnp.zeros((LANES,), jnp.float32)
        for rr in range(LANES):
            r = half * LANES + rr
            v = wide_v[pl.ds(r * LANES, LANES)]
            s = v[0]
            for p in range(1, LANES):
                s = s + v[p]
            tot = jnp.where(lane_ids == rr, s, tot)
        out_v[0, pl.ds(half * LANES, LANES)] = tot
    pltpu.sync_copy(out_v, out_hbm.at[wid])


def _tc_suffix_body(state_ref, vals_ref, acc_ref):
    c = pl.program_id(0)

    @pl.when(c == 0)
    def _():
        acc_ref[...] = jnp.zeros_like(acc_ref)

    prod = state_ref[...] * vals_ref[...]

    @pl.when(c == TC_NBLK - 1)
    def _():
        col0 = (TC_FIRST_BLK + c) * TCB
        cols = col0 + lax.broadcasted_iota(jnp.int32, (1, TCB), 1)
        p = jnp.where(cols < K, prod, 0.0)
        acc_ref[...] += jnp.sum(
            p.reshape(BATCH, TCB // 128, 128), axis=1)

    @pl.when(c < TC_NBLK - 1)
    def _():
        acc_ref[...] += jnp.sum(
            prod.reshape(BATCH, TCB // 128, 128), axis=1)


def _tc_combine_body(sc_ref, acc_ref, out_ref):
    out_ref[...] = sc_ref[...] + jnp.sum(acc_ref[...], axis=1,
                                         keepdims=True)


@jax.jit
def _matvec(state, values):
    mesh = plsc.VectorSubcoreMesh(
        core_axis_name="c", subcore_axis_name="s",
        num_cores=2, num_subcores=16,
    )
    sc_fn = pl.kernel(
        _sc_body,
        out_type=jax.ShapeDtypeStruct((NWORKERS, 1, ROWS_PER_W), jnp.float32),
        mesh=mesh,
        scratch_types=[
            pltpu.VMEM((PR, CH), jnp.float32),
            pltpu.VMEM((PR, CH), jnp.float32),
            pltpu.VMEM((CH,), jnp.float32),
            pltpu.VMEM((CH,), jnp.float32),
            pltpu.VMEM((ROWS_PER_W * LANES,), jnp.float32),
            pltpu.VMEM((1, ROWS_PER_W), jnp.float32),
            pltpu.SemaphoreType.DMA,
            pltpu.SemaphoreType.DMA,
            pltpu.SemaphoreType.DMA,
            pltpu.SemaphoreType.DMA,
        ],
    )
    vals_flat = values.reshape(K)
    out_sc = sc_fn(state, vals_flat[:KSC])
    part_sc = out_sc.reshape(BATCH, 1)

    acc_tc = pl.pallas_call(
        _tc_suffix_body,
        grid=(TC_NBLK,),
        in_specs=[
            pl.BlockSpec((BATCH, TCB), lambda c: (0, TC_FIRST_BLK + c)),
            pl.BlockSpec((1, TCB), lambda c: (0, TC_FIRST_BLK + c)),
        ],
        out_specs=pl.BlockSpec((BATCH, 128), lambda c: (0, 0)),
        out_shape=jax.ShapeDtypeStruct((BATCH, 128), jnp.float32),
    )(state, values.reshape(1, K))

    out = pl.pallas_call(
        _tc_combine_body,
        out_shape=jax.ShapeDtypeStruct((BATCH, 1), jnp.float32),
    )(part_sc, acc_tc)
    return out


def kernel(state, values):
    return _matvec(state, values)


# speedup vs baseline: 8.5400x; 3.5288x over previous
import functools
import jax
import jax.numpy as jnp
from jax import lax
from jax.experimental import pallas as pl
from jax.experimental.pallas import tpu as pltpu
from jax.experimental.pallas import tpu_sc as plsc

BATCH = 1024
K = 100000
LANES = 16
SC_CHK = 192
SC_NCHUNK = 16
SC_QUARTERS = 4
KSC = SC_QUARTERS * SC_NCHUNK * SC_CHK
TCB_K = 2048
TC_FIRST = KSC // TCB_K
K_FULL = KSC + ((K - KSC) // TCB_K) * TCB_K
TC_NBLK = (K_FULL - KSC) // TCB_K
TAIL = K - K_FULL


def _sc_body(stateT_hbm, values_hbm, out_hbm, buf0, buf1, vb0, vb1,
             out_v, sem0, sem1, vsem0, vsem1):
    wid = lax.axis_index("s") * 2 + lax.axis_index("c")
    q = wid // 8
    l = wid % 8
    kbase = q * SC_NCHUNK * SC_CHK
    lane0 = l * 128

    def start_state(c, buf, sem):
        k0 = pl.multiple_of(kbase + c * SC_CHK, 8)
        l0 = pl.multiple_of(lane0, 128)
        pltpu.async_copy(
            stateT_hbm.at[pl.ds(k0, SC_CHK), pl.ds(l0, 128)], buf, sem)

    def start_vals(c, vb, vsem):
        k0 = pl.multiple_of(kbase + c * SC_CHK, 8)
        pltpu.async_copy(values_hbm.at[pl.ds(k0, SC_CHK)], vb, vsem)

    def wait_state(buf, sem):
        pltpu.make_async_copy(
            stateT_hbm.at[pl.ds(0, SC_CHK), pl.ds(0, 128)], buf, sem).wait()

    def wait_vals(vb, vsem):
        pltpu.make_async_copy(values_hbm.at[pl.ds(0, SC_CHK)], vb, vsem).wait()

    def fma_chunk(buf, vb, accs):
        def inner(g, accs):
            vvec = vb[pl.ds(g * LANES, LANES)]
            for p in range(LANES):
                s = vvec[p]
                row = g * LANES + p
                accs = tuple(
                    accs[i] + buf[row, pl.ds(i * LANES, LANES)] * s
                    for i in range(8)
                )
            return accs
        return lax.fori_loop(0, SC_CHK // LANES, inner, accs)

    start_state(0, buf0, sem0)
    start_vals(0, vb0, vsem0)

    def pair_body(i, accs):
        c = i * 2
        start_state(c + 1, buf1, sem1)
        start_vals(c + 1, vb1, vsem1)
        wait_state(buf0, sem0)
        wait_vals(vb0, vsem0)
        accs = fma_chunk(buf0, vb0, accs)
        start_state(c + 2, buf0, sem0)
        start_vals(c + 2, vb0, vsem0)
        wait_state(buf1, sem1)
        wait_vals(vb1, vsem1)
        return fma_chunk(buf1, vb1, accs)

    zero = jnp.zeros((LANES,), jnp.float32)
    accs = lax.fori_loop(0, SC_NCHUNK // 2 - 1, pair_body, (zero,) * 8)

    start_state(SC_NCHUNK - 1, buf1, sem1)
    start_vals(SC_NCHUNK - 1, vb1, vsem1)
    wait_state(buf0, sem0)
    wait_vals(vb0, vsem0)
    accs = fma_chunk(buf0, vb0, accs)
    wait_state(buf1, sem1)
    wait_vals(vb1, vsem1)
    accs = fma_chunk(buf1, vb1, accs)

    for i in range(8):
        out_v[pl.ds(i * LANES, LANES)] = accs[i]
    pltpu.sync_copy(out_v, out_hbm.at[q, l])


def _tc_main_body(vals_ref, state_ref, acc_ref):
    c = pl.program_id(0)

    @pl.when(c == 0)
    def _():
        acc_ref[...] = jnp.zeros_like(acc_ref)

    acc_ref[...] += jnp.dot(vals_ref[...], state_ref[...],
                            preferred_element_type=jnp.float32)


def _tc_combine_body(acc_ref, vtail_ref, stail_ref, sc_ref, out_ref):
    tail = jnp.dot(vtail_ref[...], stail_ref[...],
                   preferred_element_type=jnp.float32)
    scp = jnp.sum(sc_ref[...], axis=0).reshape(1, BATCH)
    out_ref[...] = acc_ref[...] + tail + scp


@jax.jit
def _matvec(state, values):
    stateT = state.T
    vals2d = values.reshape(1, K)
    vals1d = values.reshape(K)

    mesh = plsc.VectorSubcoreMesh(
        core_axis_name="c", subcore_axis_name="s",
        num_cores=2, num_subcores=16,
    )
    sc_fn = pl.kernel(
        _sc_body,
        out_type=jax.ShapeDtypeStruct((SC_QUARTERS, 8, 128), jnp.float32),
        mesh=mesh,
        scratch_types=[
            pltpu.VMEM((SC_CHK, 128), jnp.float32),
            pltpu.VMEM((SC_CHK, 128), jnp.float32),
            pltpu.VMEM((SC_CHK,), jnp.float32),
            pltpu.VMEM((SC_CHK,), jnp.float32),
            pltpu.VMEM((128,), jnp.float32),
            pltpu.SemaphoreType.DMA,
            pltpu.SemaphoreType.DMA,
            pltpu.SemaphoreType.DMA,
            pltpu.SemaphoreType.DMA,
        ],
    )
    part_sc = sc_fn(stateT, vals1d[:KSC])
    part_sc = part_sc.reshape(SC_QUARTERS, BATCH)

    acc_tc = pl.pallas_call(
        _tc_main_body,
        grid=(TC_NBLK,),
        in_specs=[
            pl.BlockSpec((1, TCB_K), lambda c: (0, TC_FIRST + c)),
            pl.BlockSpec((TCB_K, BATCH), lambda c: (TC_FIRST + c, 0)),
        ],
        out_specs=pl.BlockSpec((1, BATCH), lambda c: (0, 0)),
        out_shape=jax.ShapeDtypeStruct((1, BATCH), jnp.float32),
    )(vals2d, stateT)

    vtail = lax.slice(vals2d, (0, K_FULL), (1, K))
    stail = lax.slice(stateT, (K_FULL, 0), (K, BATCH))
    out = pl.pallas_call(
        _tc_combine_body,
        out_shape=jax.ShapeDtypeStruct((1, BATCH), jnp.float32),
    )(acc_tc, vtail, stail, part_sc)
    return out.reshape(BATCH, 1)


def kernel(state, values):
    return _matvec(state, values)


# speedup vs baseline: 8.6584x; 1.0139x over previous
import functools
import jax
import jax.numpy as jnp
from jax import lax
from jax.experimental import pallas as pl
from jax.experimental.pallas import tpu as pltpu
from jax.experimental.pallas import tpu_sc as plsc

BATCH = 1024
K = 100000
LANES = 16
SC_CHK = 192
SC_NCHUNK = 8
SC_QUARTERS = 4
KSC = SC_QUARTERS * SC_NCHUNK * SC_CHK
TCB_K = 2048
TC_FIRST = KSC // TCB_K
K_FULL = KSC + ((K - KSC) // TCB_K) * TCB_K
TC_NBLK = (K_FULL - KSC) // TCB_K
TAIL = K - K_FULL


def _sc_body(stateT_hbm, values_hbm, out_hbm, buf0, buf1, vb0, vb1,
             out_v, sem0, sem1, vsem0, vsem1):
    wid = lax.axis_index("s") * 2 + lax.axis_index("c")
    q = wid // 8
    l = wid % 8
    kbase = q * SC_NCHUNK * SC_CHK
    lane0 = l * 128

    def start_state(c, buf, sem):
        k0 = pl.multiple_of(kbase + c * SC_CHK, 8)
        l0 = pl.multiple_of(lane0, 128)
        pltpu.async_copy(
            stateT_hbm.at[pl.ds(k0, SC_CHK), pl.ds(l0, 128)], buf, sem)

    def start_vals(c, vb, vsem):
        k0 = pl.multiple_of(kbase + c * SC_CHK, 8)
        pltpu.async_copy(values_hbm.at[pl.ds(k0, SC_CHK)], vb, vsem)

    def wait_state(buf, sem):
        pltpu.make_async_copy(
            stateT_hbm.at[pl.ds(0, SC_CHK), pl.ds(0, 128)], buf, sem).wait()

    def wait_vals(vb, vsem):
        pltpu.make_async_copy(values_hbm.at[pl.ds(0, SC_CHK)], vb, vsem).wait()

    def fma_chunk(buf, vb, accs):
        def inner(g, accs):
            vvec = vb[pl.ds(g * LANES, LANES)]
            for p in range(LANES):
                s = vvec[p]
                row = g * LANES + p
                accs = tuple(
                    accs[i] + buf[row, pl.ds(i * LANES, LANES)] * s
                    for i in range(8)
                )
            return accs
        return lax.fori_loop(0, SC_CHK // LANES, inner, accs)

    start_state(0, buf0, sem0)
    start_vals(0, vb0, vsem0)

    def pair_body(i, accs):
        c = i * 2
        start_state(c + 1, buf1, sem1)
        start_vals(c + 1, vb1, vsem1)
        wait_state(buf0, sem0)
        wait_vals(vb0, vsem0)
        accs = fma_chunk(buf0, vb0, accs)
        start_state(c + 2, buf0, sem0)
        start_vals(c + 2, vb0, vsem0)
        wait_state(buf1, sem1)
        wait_vals(vb1, vsem1)
        return fma_chunk(buf1, vb1, accs)

    zero = jnp.zeros((LANES,), jnp.float32)
    accs = lax.fori_loop(0, SC_NCHUNK // 2 - 1, pair_body, (zero,) * 8)

    start_state(SC_NCHUNK - 1, buf1, sem1)
    start_vals(SC_NCHUNK - 1, vb1, vsem1)
    wait_state(buf0, sem0)
    wait_vals(vb0, vsem0)
    accs = fma_chunk(buf0, vb0, accs)
    wait_state(buf1, sem1)
    wait_vals(vb1, vsem1)
    accs = fma_chunk(buf1, vb1, accs)

    for i in range(8):
        out_v[pl.ds(i * LANES, LANES)] = accs[i]
    pltpu.sync_copy(out_v, out_hbm.at[q, l])


def _tc_main_body(vals_ref, state_ref, acc_ref):
    c = pl.program_id(0)

    @pl.when(c == 0)
    def _():
        acc_ref[...] = jnp.zeros_like(acc_ref)

    acc_ref[...] += jnp.dot(vals_ref[...], state_ref[...],
                            preferred_element_type=jnp.float32)


def _tc_combine_body(acc_ref, vtail_ref, stail_ref, sc_ref, out_ref):
    tail = jnp.dot(vtail_ref[...], stail_ref[...],
                   preferred_element_type=jnp.float32)
    scp = jnp.sum(sc_ref[...], axis=0).reshape(1, BATCH)
    out_ref[...] = acc_ref[...] + tail + scp


@jax.jit
def _matvec(state, values):
    stateT = state.T
    vals2d = values.reshape(1, K)
    vals1d = values.reshape(K)

    mesh = plsc.VectorSubcoreMesh(
        core_axis_name="c", subcore_axis_name="s",
        num_cores=2, num_subcores=16,
    )
    sc_fn = pl.kernel(
        _sc_body,
        out_type=jax.ShapeDtypeStruct((SC_QUARTERS, 8, 128), jnp.float32),
        mesh=mesh,
        scratch_types=[
            pltpu.VMEM((SC_CHK, 128), jnp.float32),
            pltpu.VMEM((SC_CHK, 128), jnp.float32),
            pltpu.VMEM((SC_CHK,), jnp.float32),
            pltpu.VMEM((SC_CHK,), jnp.float32),
            pltpu.VMEM((128,), jnp.float32),
            pltpu.SemaphoreType.DMA,
            pltpu.SemaphoreType.DMA,
            pltpu.SemaphoreType.DMA,
            pltpu.SemaphoreType.DMA,
        ],
    )
    part_sc = sc_fn(stateT, vals1d[:KSC])
    part_sc = part_sc.reshape(SC_QUARTERS, BATCH)

    acc_tc = pl.pallas_call(
        _tc_main_body,
        grid=(TC_NBLK,),
        in_specs=[
            pl.BlockSpec((1, TCB_K), lambda c: (0, TC_FIRST + c)),
            pl.BlockSpec((TCB_K, BATCH), lambda c: (TC_FIRST + c, 0)),
        ],
        out_specs=pl.BlockSpec((1, BATCH), lambda c: (0, 0)),
        out_shape=jax.ShapeDtypeStruct((1, BATCH), jnp.float32),
    )(vals2d, stateT)

    vtail = lax.slice(vals2d, (0, K_FULL), (1, K))
    stail = lax.slice(stateT, (K_FULL, 0), (K, BATCH))
    out = pl.pallas_call(
        _tc_combine_body,
        out_shape=jax.ShapeDtypeStruct((1, BATCH), jnp.float32),
    )(acc_tc, vtail, stail, part_sc)
    return out.reshape(BATCH, 1)


def kernel(state, values):
    return _matvec(state, values)
